# Initial kernel scaffold; baseline (speedup 1.0000x reference)
import jax
import jax.numpy as jnp
from jax.experimental import pallas as pl


def kernel(features, rows, cols, W1, b1, W2, b2):
    raise NotImplementedError("write your pallas kernel here")


# speedup vs baseline: 11.3546x; 11.3546x over previous
import functools

import jax
import jax.numpy as jnp
from jax import lax
from jax.experimental import pallas as pl
from jax.experimental.pallas import tpu as pltpu
from jax.experimental.pallas import tpu_sc as plsc

N = 10000
D = 128
NC, NS, L = 2, 16, 16
NW = NC * NS
B = 128
N_PAD = 10240
RPT = N_PAD // NS


def _sc_agg_body(with_deg, *refs):
    if with_deg:
        (x_hbm, cols_hbm, rows_hbm, part_hbm, deg_hbm,
         colsv, rowsv, gbuf, zrow, zdeg, onesv, acc, dacc, sem) = refs
    else:
        (x_hbm, cols_hbm, rows_hbm, part_hbm,
         colsv, rowsv, gbuf, zrow, acc, sem) = refs

    c = lax.axis_index("c")
    s = lax.axis_index("s")
    wid = s * NC + c
    row0 = s * RPT
    k_blocks = colsv.shape[0]

    pltpu.sync_copy(cols_hbm.at[wid], colsv)
    pltpu.sync_copy(rows_hbm.at[wid], rowsv)

    zeros16 = jnp.zeros((L,), jnp.float32)
    for r in range(16):
        for j in range(D // L):
            zrow[r, pl.ds(j * L, L)] = zeros16

    def zero_blk(k, _):
        pltpu.sync_copy(zrow, acc.at[pl.ds(row0 + k * 16, 16)])
        return _
    lax.fori_loop(0, RPT // 16, zero_blk, None)

    if with_deg:
        for i in range(RPT // L):
            zdeg[pl.ds(i * L, L)] = zeros16
        pltpu.sync_copy(zdeg, dacc.at[pl.ds(row0, RPT)])
        ones16 = jnp.ones((L,), jnp.float32)
        for i in range(B // L):
            onesv[pl.ds(i * L, L)] = ones16

    plsc.subcore_barrier()

    def blk(j, _):
        pltpu.async_copy(x_hbm.at[colsv.at[j]], gbuf, sem).wait()
        pltpu.sync_copy(gbuf, acc.at[rowsv.at[j]], add=True)
        if with_deg:
            pltpu.sync_copy(onesv, dacc.at[rowsv.at[j]], add=True)
        return _
    lax.fori_loop(0, k_blocks, blk, None)

    plsc.subcore_barrier()

    pltpu.sync_copy(acc.at[pl.ds(row0, RPT)],
                    part_hbm.at[c, pl.ds(row0, RPT)])
    if with_deg:
        pltpu.sync_copy(dacc.at[pl.ds(row0, RPT)],
                        deg_hbm.at[c, pl.ds(row0, RPT)])


def _make_sc_agg(k_blocks, with_deg):
    mesh = plsc.VectorSubcoreMesh(core_axis_name="c", subcore_axis_name="s",
                                  num_cores=NC, num_subcores=NS)
    out_type = [jax.ShapeDtypeStruct((NC, N_PAD, D), jnp.float32)]
    scratch = [
        pltpu.VMEM((k_blocks, B), jnp.int32),
        pltpu.VMEM((k_blocks, B), jnp.int32),
        pltpu.VMEM((B, D), jnp.float32),
        pltpu.VMEM((16, D), jnp.float32),
    ]
    if with_deg:
        out_type.append(jax.ShapeDtypeStruct((NC, N_PAD), jnp.float32))
        scratch += [
            pltpu.VMEM((RPT,), jnp.float32),
            pltpu.VMEM((B,), jnp.float32),
        ]
    scratch.append(pltpu.VMEM_SHARED((N_PAD, D), jnp.float32))
    if with_deg:
        scratch.append(pltpu.VMEM_SHARED((N_PAD,), jnp.float32))
    scratch.append(pltpu.SemaphoreType.DMA)
    return pl.kernel(
        functools.partial(_sc_agg_body, with_deg),
        out_type=tuple(out_type),
        mesh=mesh,
        scratch_types=scratch,
    )


def _dense_body(x_ref, p0_ref, p1_ref, d0_ref, d1_ref,
                wt_ref, wb_ref, b_ref, o_ref):
    deg = jnp.maximum(d0_ref[...] + d1_ref[...], 1.0)
    neigh = (p0_ref[...] + p1_ref[...]) / deg
    acc = jnp.dot(x_ref[...], wt_ref[...], preferred_element_type=jnp.float32)
    acc = acc + jnp.dot(neigh, wb_ref[...], preferred_element_type=jnp.float32)
    o_ref[...] = jnp.maximum(acc + b_ref[...], 0.0)


def _dense(x, p0, p1, d0, d1, wt, wb, b):
    R = 512
    grid = (N_PAD // R,)
    return pl.pallas_call(
        _dense_body,
        grid=grid,
        in_specs=[
            pl.BlockSpec((R, D), lambda i: (i, 0)),
            pl.BlockSpec((R, D), lambda i: (i, 0)),
            pl.BlockSpec((R, D), lambda i: (i, 0)),
            pl.BlockSpec((R, 1), lambda i: (i, 0)),
            pl.BlockSpec((R, 1), lambda i: (i, 0)),
            pl.BlockSpec((D, D), lambda i: (0, 0)),
            pl.BlockSpec((D, D), lambda i: (0, 0)),
            pl.BlockSpec((1, D), lambda i: (0, 0)),
        ],
        out_specs=pl.BlockSpec((R, D), lambda i: (i, 0)),
        out_shape=jax.ShapeDtypeStruct((N_PAD, D), jnp.float32),
    )(x, p0, p1, d0, d1, wt, wb, b)


def kernel(features, rows, cols, W1, b1, W2, b2):
    E = rows.shape[0]
    k_blocks = -(-E // (NW * B))
    e_pad = NW * k_blocks * B

    xpad = jnp.zeros((N_PAD, D), jnp.float32).at[:N, :].set(features)

    pad = e_pad - E
    i = jnp.arange(pad, dtype=jnp.int32)
    cols_p = jnp.concatenate([cols, i % N])
    rows_p = jnp.concatenate([rows, N + i % (N_PAD - N)])
    cols_r = cols_p.reshape(NW, k_blocks, B)
    rows_r = rows_p.reshape(NW, k_blocks, B)

    agg1 = _make_sc_agg(k_blocks, with_deg=True)
    agg2 = _make_sc_agg(k_blocks, with_deg=False)

    part1, degp = agg1(xpad, cols_r, rows_r)
    d0 = degp[0][:, None]
    d1 = degp[1][:, None]
    w1t, w1b = W1[:D], W1[D:]
    w2t, w2b = W2[:D], W2[D:]

    h1 = _dense(xpad, part1[0], part1[1], d0, d1, w1t, w1b, b1[None, :])
    (part2,) = agg2(h1, cols_r, rows_r)
    h2 = _dense(h1, part2[0], part2[1], d0, d1, w2t, w2b, b2[None, :])
    return h2[:N]
